# baseline (speedup 1.0000x reference)
import functools

import jax
import jax.numpy as jnp
from jax import lax
from jax.experimental import pallas as pl
from jax.experimental.pallas import tpu as pltpu
from jax.experimental.pallas import tpu_sc as plsc

N_DRUG = 572
D = 64
K = 128
N_REL = 100

NW = 32
CHUNK = 128
CPW = 18
NBUF = 4
ROWS_PAD = NW * CPW * CHUNK
ROWS_PER_W = CPW * CHUNK

BN_DRUGS = 8
GRID = (N_DRUG + BN_DRUGS - 1) // BN_DRUGS

_PREC = jax.lax.Precision.DEFAULT


def _sc_gather_body(idx_hbm, tab_hbm, out_hbm, idx_v, bufs, sems):
    wid = lax.axis_index("s") * 2 + lax.axis_index("c")
    pltpu.sync_copy(idx_hbm.at[wid], idx_v)
    base = wid * ROWS_PER_W
    cops = [None] * NBUF
    for j in range(NBUF - 1):
        cops[j] = pltpu.async_copy(tab_hbm.at[idx_v.at[j]], bufs[j], sems[j])
    for j in range(CPW):
        nxt = j + NBUF - 1
        if nxt < CPW:
            cops[nxt % NBUF] = pltpu.async_copy(
                tab_hbm.at[idx_v.at[nxt]], bufs[nxt % NBUF], sems[nxt % NBUF])
        cops[j % NBUF].wait()
        pltpu.sync_copy(bufs[j % NBUF],
                        out_hbm.at[pl.ds(base + j * CHUNK, CHUNK)])


def _sc_gather(idx_pad, ent_table):
    mesh = plsc.VectorSubcoreMesh(core_axis_name="c", subcore_axis_name="s")

    def body(idx_hbm, tab_hbm, out_hbm, *scratch):
        _sc_gather_body(idx_hbm, tab_hbm, out_hbm, scratch[0],
                        scratch[1 : 1 + NBUF], scratch[1 + NBUF :])

    f = functools.partial(
        pl.kernel,
        mesh=mesh,
        out_type=jax.ShapeDtypeStruct((ROWS_PAD, D), jnp.float32),
        compiler_params=pltpu.CompilerParams(use_tc_tiling_on_sc=False),
        scratch_types=(
            [pltpu.VMEM((CPW, CHUNK), jnp.int32)]
            + [pltpu.VMEM((CHUNK, D), jnp.float32)] * NBUF
            + [pltpu.SemaphoreType.DMA] * NBUF
        ),
    )(body)
    return f(idx_pad, ent_table)


def _scores_body(demb, rel, w1, w2, b1, b2, rtab, pout):
    b2s = jnp.sum(b2[:], axis=1, keepdims=True)
    iota2 = lax.broadcasted_iota(jnp.int32, (128, K), 0)
    rt = rtab[:]
    b1v = b1[:]
    scs = []
    for i in range(BN_DRUGS):
        ids = rel[i : i + 1, :]
        ohT = (iota2 == ids).astype(jnp.float32)
        re = lax.dot_general(ohT, rt, (((0,), (0,)), ((), ())),
                             precision=_PREC,
                             preferred_element_type=jnp.float32)
        dr = re * demb[i : i + 1, :]
        h = jnp.maximum(
            jnp.dot(dr, w1[i], precision=_PREC,
                    preferred_element_type=jnp.float32) + b1v, 0.0)
        hw = jnp.dot(h, w2[i], precision=_PREC,
                     preferred_element_type=jnp.float32)
        scs.append(jnp.sum(hw, axis=1, keepdims=True) + b2s)
    sc_all = jnp.concatenate(scs, axis=1)
    m = jnp.max(sc_all, axis=0, keepdims=True)
    e = jnp.exp(sc_all - m)
    pout[:] = e / jnp.sum(e, axis=0, keepdims=True)


def _agg_body(p_ref, ent, demb, lw, lb, out):
    p_all = p_ref[:]
    for i in range(BN_DRUGS):
        p_i = p_all[:, i : i + 1]
        went = jnp.sum(p_i * ent[pl.ds(i * K, K), :], axis=0,
                       keepdims=True)
        out[i : i + 1, :] = went
    wb = out[:]
    x = (jnp.dot(wb, lw[0:D, :], precision=_PREC,
                 preferred_element_type=jnp.float32)
         + jnp.dot(demb[:], lw[D : 2 * D, :], precision=_PREC,
                   preferred_element_type=jnp.float32)
         + lb[:])
    out[:] = jnp.maximum(x, 0.0)


def _bn_body(x_ref, gamma, beta, out):
    x = x_ref[:]
    mean = jnp.mean(x, axis=0, keepdims=True)
    var = jnp.mean((x - mean) ** 2, axis=0, keepdims=True)
    out[:] = (x - mean) * lax.rsqrt(var + 1e-5) * gamma[:] + beta[:]


def _tc_scores(drug_emb, rel, W1, W2, b1, b2, rtab_pad):
    return pl.pallas_call(
        _scores_body,
        grid=(GRID,),
        in_specs=[
            pl.BlockSpec((BN_DRUGS, D), lambda i: (i, 0)),
            pl.BlockSpec((BN_DRUGS, K), lambda i: (i, 0)),
            pl.BlockSpec((BN_DRUGS, D, D), lambda i: (i, 0, 0)),
            pl.BlockSpec((BN_DRUGS, D, D), lambda i: (i, 0, 0)),
            pl.BlockSpec((K, D), lambda i: (0, 0)),
            pl.BlockSpec((K, D), lambda i: (0, 0)),
            pl.BlockSpec((128, D), lambda i: (0, 0)),
        ],
        out_specs=pl.BlockSpec((K, BN_DRUGS), lambda i: (i, 0)),
        out_shape=jax.ShapeDtypeStruct((GRID * K, BN_DRUGS), jnp.float32),
    )(drug_emb, rel, W1, W2, b1, b2, rtab_pad)


def _tc_agg(p_cols, ent_rows, drug_emb, lin_w, lin_b2):
    return pl.pallas_call(
        _agg_body,
        grid=(GRID,),
        in_specs=[
            pl.BlockSpec((K, BN_DRUGS), lambda i: (i, 0)),
            pl.BlockSpec((BN_DRUGS * K, D), lambda i: (i, 0)),
            pl.BlockSpec((BN_DRUGS, D), lambda i: (i, 0)),
            pl.BlockSpec((2 * D, D), lambda i: (0, 0)),
            pl.BlockSpec((1, D), lambda i: (0, 0)),
        ],
        out_specs=pl.BlockSpec((BN_DRUGS, D), lambda i: (i, 0)),
        out_shape=jax.ShapeDtypeStruct((N_DRUG, D), jnp.float32),
    )(p_cols, ent_rows, drug_emb, lin_w, lin_b2)


def _tc_bn(xr, gamma2, beta2):
    return pl.pallas_call(
        _bn_body,
        in_specs=[
            pl.BlockSpec((N_DRUG, D), lambda: (0, 0)),
            pl.BlockSpec((1, D), lambda: (0, 0)),
            pl.BlockSpec((1, D), lambda: (0, 0)),
        ],
        out_specs=pl.BlockSpec((N_DRUG, D), lambda: (0, 0)),
        out_shape=jax.ShapeDtypeStruct((N_DRUG, D), jnp.float32),
    )(xr, gamma2, beta2)


def kernel(gnn3_embedding, gnn2_embedding, gnn1_embedding, idx, drug_name,
           adj_tail, adj_relation, drug_table, rela_table, ent_table,
           W1, b1, W2, b2, lin_w, lin_b, bn_gamma, bn_beta):
    drug_emb = jnp.take(drug_table, drug_name, axis=0)
    idx_flat = adj_tail.reshape(-1)
    idx_pad = jnp.pad(idx_flat, (0, ROWS_PAD - N_DRUG * K)).reshape(
        NW, CPW, CHUNK)
    ent_rows = _sc_gather(idx_pad, ent_table)
    rtab_pad = jnp.pad(rela_table, ((0, 128 - N_REL), (0, 0)))
    p_cols = _tc_scores(drug_emb, adj_relation, W1, W2, b1, b2, rtab_pad)
    xr = _tc_agg(p_cols, ent_rows, drug_emb, lin_w, lin_b.reshape(1, D))
    drug_f = _tc_bn(xr, bn_gamma.reshape(1, D), bn_beta.reshape(1, D))
    return (drug_f, gnn3_embedding, gnn2_embedding, gnn1_embedding, idx)


# speedup vs baseline: 1.1140x; 1.1140x over previous
import functools

import jax
import jax.numpy as jnp
from jax import lax
from jax.experimental import pallas as pl
from jax.experimental.pallas import tpu as pltpu
from jax.experimental.pallas import tpu_sc as plsc

N_DRUG = 572
D = 64
K = 128
N_REL = 100

NW = 32
CHUNK = 128
CPW = 18
NBUF = 4
ROWS_PAD = NW * CPW * CHUNK
ROWS_PER_W = CPW * CHUNK

BN_DRUGS = 8
GRID = (N_DRUG + BN_DRUGS - 1) // BN_DRUGS

_PREC = jax.lax.Precision.DEFAULT


def _sc_gather_body(idx_hbm, tab_hbm, out_hbm, idx_v, bufs, sems):
    wid = lax.axis_index("s") * 2 + lax.axis_index("c")
    pltpu.sync_copy(idx_hbm.at[wid], idx_v)
    base = wid * ROWS_PER_W
    cops = [None] * NBUF
    for j in range(NBUF - 1):
        cops[j] = pltpu.async_copy(tab_hbm.at[idx_v.at[j]], bufs[j], sems[j])
    for j in range(CPW):
        nxt = j + NBUF - 1
        if nxt < CPW:
            cops[nxt % NBUF] = pltpu.async_copy(
                tab_hbm.at[idx_v.at[nxt]], bufs[nxt % NBUF], sems[nxt % NBUF])
        cops[j % NBUF].wait()
        pltpu.sync_copy(bufs[j % NBUF],
                        out_hbm.at[pl.ds(base + j * CHUNK, CHUNK)])


def _sc_gather(idx_pad, ent_table):
    mesh = plsc.VectorSubcoreMesh(core_axis_name="c", subcore_axis_name="s")

    def body(idx_hbm, tab_hbm, out_hbm, *scratch):
        _sc_gather_body(idx_hbm, tab_hbm, out_hbm, scratch[0],
                        scratch[1 : 1 + NBUF], scratch[1 + NBUF :])

    f = functools.partial(
        pl.kernel,
        mesh=mesh,
        out_type=jax.ShapeDtypeStruct((ROWS_PAD, D), jnp.float32),
        compiler_params=pltpu.CompilerParams(use_tc_tiling_on_sc=False),
        scratch_types=(
            [pltpu.VMEM((CPW, CHUNK), jnp.int32)]
            + [pltpu.VMEM((CHUNK, D), jnp.float32)] * NBUF
            + [pltpu.SemaphoreType.DMA] * NBUF
        ),
    )(body)
    return f(idx_pad, ent_table)


def _main_body(demb, rel, w1, w2, ent, b1, b2, rtab, lw, lb, out):
    b2s = jnp.sum(b2[:], axis=1, keepdims=True)
    iota2 = lax.broadcasted_iota(jnp.int32, (128, K), 0)
    rt = rtab[:]
    b1v = b1[:]
    scs = []
    for i in range(BN_DRUGS):
        ids = rel[i : i + 1, :]
        ohT = (iota2 == ids).astype(jnp.float32)
        re = lax.dot_general(ohT, rt, (((0,), (0,)), ((), ())),
                             precision=_PREC,
                             preferred_element_type=jnp.float32)
        dr = re * demb[i : i + 1, :]
        h = jnp.maximum(
            jnp.dot(dr, w1[i], precision=_PREC,
                    preferred_element_type=jnp.float32) + b1v, 0.0)
        hw = jnp.dot(h, w2[i], precision=_PREC,
                     preferred_element_type=jnp.float32)
        scs.append(jnp.sum(hw, axis=1, keepdims=True) + b2s)
    sc_all = jnp.concatenate(scs, axis=1)
    m = jnp.max(sc_all, axis=0, keepdims=True)
    e = jnp.exp(sc_all - m)
    p_all = e / jnp.sum(e, axis=0, keepdims=True)
    for i in range(BN_DRUGS):
        p_i = p_all[:, i : i + 1]
        went = jnp.sum(p_i * ent[pl.ds(i * K, K), :], axis=0,
                       keepdims=True)
        out[i : i + 1, :] = went
    wb = out[:]
    x = (jnp.dot(wb, lw[0:D, :], precision=_PREC,
                 preferred_element_type=jnp.float32)
         + jnp.dot(demb[:], lw[D : 2 * D, :], precision=_PREC,
                   preferred_element_type=jnp.float32)
         + lb[:])
    out[:] = jnp.maximum(x, 0.0)


def _tc_main(drug_emb, rel, W1, W2, ent_rows, b1, b2, rtab_pad, lin_w,
             lin_b2):
    return pl.pallas_call(
        _main_body,
        grid=(GRID,),
        in_specs=[
            pl.BlockSpec((BN_DRUGS, D), lambda i: (i, 0)),
            pl.BlockSpec((BN_DRUGS, K), lambda i: (i, 0)),
            pl.BlockSpec((BN_DRUGS, D, D), lambda i: (i, 0, 0)),
            pl.BlockSpec((BN_DRUGS, D, D), lambda i: (i, 0, 0)),
            pl.BlockSpec((BN_DRUGS * K, D), lambda i: (i, 0)),
            pl.BlockSpec((K, D), lambda i: (0, 0)),
            pl.BlockSpec((K, D), lambda i: (0, 0)),
            pl.BlockSpec((128, D), lambda i: (0, 0)),
            pl.BlockSpec((2 * D, D), lambda i: (0, 0)),
            pl.BlockSpec((1, D), lambda i: (0, 0)),
        ],
        out_specs=pl.BlockSpec((BN_DRUGS, D), lambda i: (i, 0)),
        out_shape=jax.ShapeDtypeStruct((N_DRUG, D), jnp.float32),
    )(drug_emb, rel, W1, W2, ent_rows, b1, b2, rtab_pad, lin_w, lin_b2)


def _bn_body(x_ref, gamma, beta, out):
    x = x_ref[:]
    mean = jnp.mean(x, axis=0, keepdims=True)
    var = jnp.mean((x - mean) ** 2, axis=0, keepdims=True)
    out[:] = (x - mean) * lax.rsqrt(var + 1e-5) * gamma[:] + beta[:]


def _tc_bn(xr, gamma2, beta2):
    return pl.pallas_call(
        _bn_body,
        in_specs=[
            pl.BlockSpec((N_DRUG, D), lambda: (0, 0)),
            pl.BlockSpec((1, D), lambda: (0, 0)),
            pl.BlockSpec((1, D), lambda: (0, 0)),
        ],
        out_specs=pl.BlockSpec((N_DRUG, D), lambda: (0, 0)),
        out_shape=jax.ShapeDtypeStruct((N_DRUG, D), jnp.float32),
    )(xr, gamma2, beta2)


def kernel(gnn3_embedding, gnn2_embedding, gnn1_embedding, idx, drug_name,
           adj_tail, adj_relation, drug_table, rela_table, ent_table,
           W1, b1, W2, b2, lin_w, lin_b, bn_gamma, bn_beta):
    drug_emb = jnp.take(drug_table, drug_name, axis=0)
    idx_flat = adj_tail.reshape(-1)
    idx_pad = jnp.pad(idx_flat, (0, ROWS_PAD - N_DRUG * K)).reshape(
        NW, CPW, CHUNK)
    ent_rows = _sc_gather(idx_pad, ent_table)
    rtab_pad = jnp.pad(rela_table, ((0, 128 - N_REL), (0, 0)))
    xr = _tc_main(drug_emb, adj_relation, W1, W2, ent_rows, b1, b2,
                  rtab_pad, lin_w, lin_b.reshape(1, D))
    drug_f = _tc_bn(xr, bn_gamma.reshape(1, D), bn_beta.reshape(1, D))
    return (drug_f, gnn3_embedding, gnn2_embedding, gnn1_embedding, idx)
